# baseline (speedup 1.0000x reference)
import functools

import jax
import jax.numpy as jnp
from jax import lax
from jax.experimental import pallas as pl
from jax.experimental.pallas import tpu as pltpu
from jax.experimental.pallas import tpu_sc as plsc

VOCAB = 1000
L = 20
B = 1024
ROWS = B * L
NC, NS = 2, 16
NW = NC * NS
B_PER_W = ROWS // NW
OROWS_PER_W = B_PER_W // L
LP = 24
OROW_PER_CHUNK = 2
CHUNK = OROW_PER_CHUNK * L
NCHUNK = OROWS_PER_W // OROW_PER_CHUNK


def _sc_gather(table, flat_idx):
    mesh = plsc.VectorSubcoreMesh(core_axis_name="c", subcore_axis_name="s")

    @functools.partial(
        pl.kernel,
        mesh=mesh,
        out_type=jax.ShapeDtypeStruct((B, L, VOCAB), jnp.float32),
        scratch_types=[
            pltpu.VMEM((OROWS_PER_W * LP,), jnp.int32),
            pltpu.VMEM((OROW_PER_CHUNK, L, VOCAB), jnp.float32),
            pltpu.VMEM((OROW_PER_CHUNK, L, VOCAB), jnp.float32),
            pltpu.SemaphoreType.DMA,
            pltpu.SemaphoreType.DMA,
            pltpu.SemaphoreType.DMA,
            pltpu.SemaphoreType.DMA,
        ],
        compiler_params=pltpu.CompilerParams(use_tc_tiling_on_sc=False),
    )
    def k(table_hbm, idx_hbm, out_hbm, idx_v, buf0, buf1, g0, g1, s0, s1):
        wid = lax.axis_index("s") * NC + lax.axis_index("c")
        obase = wid * OROWS_PER_W
        pltpu.sync_copy(idx_hbm.at[pl.ds(obase * LP, OROWS_PER_W * LP)], idx_v)

        bufs = (buf0, buf1)
        gsems = (g0, g1)
        ssems = (s0, s1)

        def gather(i):
            descs = []
            for j in range(OROW_PER_CHUNK):
                descs.append(pltpu.async_copy(
                    table_hbm.at[idx_v.at[pl.ds((i * OROW_PER_CHUNK + j) * LP, L)]],
                    bufs[i % 2].at[j],
                    gsems[i % 2],
                ))
            return descs

        def store(i):
            return pltpu.async_copy(
                bufs[i % 2],
                out_hbm.at[pl.ds(obase + i * OROW_PER_CHUNK, OROW_PER_CHUNK)],
                ssems[i % 2],
            )

        def wait_gather(descs):
            for d in descs:
                d.wait()

        gd = [None] * NCHUNK
        sd = [None] * NCHUNK
        gd[0] = gather(0)
        gd[1] = gather(1)
        wait_gather(gd[0])
        sd[0] = store(0)
        for i in range(1, NCHUNK):
            sd[i - 1].wait()
            if i + 1 < NCHUNK:
                gd[i + 1] = gather(i + 1)
            wait_gather(gd[i])
            sd[i] = store(i)
        sd[NCHUNK - 1].wait()

    return k(table, flat_idx)


def kernel(inputs, table):
    idx = jnp.pad(inputs.astype(jnp.int32), ((0, 0), (0, LP - L)))
    out = _sc_gather(table, idx.reshape(-1))
    return out.reshape(B, L * VOCAB)


# speedup vs baseline: 1.0916x; 1.0916x over previous
import functools

import jax
import jax.numpy as jnp
from jax import lax
from jax.experimental import pallas as pl
from jax.experimental.pallas import tpu as pltpu
from jax.experimental.pallas import tpu_sc as plsc

VOCAB = 1000
L = 20
B = 1024
NC, NS = 2, 16
NW = NC * NS
B_PER_W = B // NW


def _sc_gather(table, idx_t):
    mesh = plsc.VectorSubcoreMesh(core_axis_name="c", subcore_axis_name="s")

    @functools.partial(
        pl.kernel,
        mesh=mesh,
        out_type=jax.ShapeDtypeStruct((B, L * VOCAB), jnp.float32),
        scratch_types=[
            pltpu.VMEM((L, B_PER_W), jnp.int32),
            pltpu.VMEM((B_PER_W, VOCAB), jnp.float32),
            pltpu.VMEM((B_PER_W, VOCAB), jnp.float32),
            pltpu.SemaphoreType.DMA,
            pltpu.SemaphoreType.DMA,
            pltpu.SemaphoreType.DMA,
            pltpu.SemaphoreType.DMA,
        ],
        compiler_params=pltpu.CompilerParams(use_tc_tiling_on_sc=False),
    )
    def k(table_hbm, idxt_hbm, out_hbm, idx_v, buf0, buf1, g0, g1, s0, s1):
        wid = lax.axis_index("s") * NC + lax.axis_index("c")
        b0 = wid * B_PER_W
        pltpu.sync_copy(idxt_hbm.at[:, pl.ds(b0, B_PER_W)], idx_v)

        bufs = (buf0, buf1)
        gsems = (g0, g1)
        ssems = (s0, s1)

        def gather(l):
            return pltpu.async_copy(
                table_hbm.at[idx_v.at[l]],
                bufs[l % 2],
                gsems[l % 2],
            )

        def store(l):
            return pltpu.async_copy(
                bufs[l % 2],
                out_hbm.at[pl.ds(b0, B_PER_W), pl.ds(l * VOCAB, VOCAB)],
                ssems[l % 2],
            )

        gd = [None] * L
        sd = [None] * L
        gd[0] = gather(0)
        gd[1] = gather(1)
        gd[0].wait()
        sd[0] = store(0)
        for l in range(1, L):
            sd[l - 1].wait()
            if l + 1 < L:
                gd[l + 1] = gather(l + 1)
            gd[l].wait()
            sd[l] = store(l)
        sd[L - 1].wait()

    return k(table, idx_t)


def kernel(inputs, table):
    idx_t = inputs.astype(jnp.int32).T
    return _sc_gather(table, idx_t)


# speedup vs baseline: 1.0934x; 1.0017x over previous
import functools

import jax
import jax.numpy as jnp
from jax import lax
from jax.experimental import pallas as pl
from jax.experimental.pallas import tpu as pltpu
from jax.experimental.pallas import tpu_sc as plsc

VOCAB = 1000
L = 20
B = 1024
ROWS = B * L
NC, NS = 2, 16
NW = NC * NS
RPW = ROWS // NW
CHUNK = 40
NCHUNK = RPW // CHUNK


def _sc_gather(table, idx_flat):
    mesh = plsc.VectorSubcoreMesh(core_axis_name="c", subcore_axis_name="s")

    @functools.partial(
        pl.kernel,
        mesh=mesh,
        out_type=jax.ShapeDtypeStruct((ROWS, VOCAB), jnp.float32),
        scratch_types=[
            pltpu.VMEM((RPW,), jnp.int32),
            pltpu.VMEM((CHUNK, VOCAB), jnp.float32),
            pltpu.VMEM((CHUNK, VOCAB), jnp.float32),
            pltpu.SemaphoreType.DMA,
            pltpu.SemaphoreType.DMA,
            pltpu.SemaphoreType.DMA,
            pltpu.SemaphoreType.DMA,
        ],
        compiler_params=pltpu.CompilerParams(use_tc_tiling_on_sc=False),
    )
    def k(table_hbm, idx_hbm, out_hbm, idx_v, buf0, buf1, g0, g1, s0, s1):
        wid = lax.axis_index("s") * NC + lax.axis_index("c")
        r0 = wid * RPW
        pltpu.sync_copy(idx_hbm.at[pl.ds(r0, RPW)], idx_v)

        bufs = (buf0, buf1)
        gsems = (g0, g1)
        ssems = (s0, s1)

        def gather(c):
            return pltpu.async_copy(
                table_hbm.at[idx_v.at[pl.ds(c * CHUNK, CHUNK)]],
                bufs[c % 2],
                gsems[c % 2],
            )

        def store(c):
            return pltpu.async_copy(
                bufs[c % 2],
                out_hbm.at[pl.ds(r0 + c * CHUNK, CHUNK)],
                ssems[c % 2],
            )

        gd = [None] * NCHUNK
        sd = [None] * NCHUNK
        gd[0] = gather(0)
        gd[1] = gather(1)
        gd[0].wait()
        sd[0] = store(0)
        for c in range(1, NCHUNK):
            sd[c - 1].wait()
            if c + 1 < NCHUNK:
                gd[c + 1] = gather(c + 1)
            gd[c].wait()
            sd[c] = store(c)
        sd[NCHUNK - 1].wait()

    return k(table, idx_flat)


def kernel(inputs, table):
    idx_flat = inputs.astype(jnp.int32).reshape(ROWS)
    return _sc_gather(table, idx_flat).reshape(B, L * VOCAB)
